# baseline (speedup 1.0000x reference)
import functools

import jax
import jax.numpy as jnp
from jax import lax
from jax.experimental import pallas as pl
from jax.experimental.pallas import tpu as pltpu
from jax.experimental.pallas import tpu_sc as plsc

B, S, D = 4, 8192, 1024
NC, NS = 2, 16
NW = NC * NS
SPW = S // NW
C = 32
NCHUNK = SPW // C
CW = C * D
U = 8

_mesh = plsc.VectorSubcoreMesh(
    core_axis_name="c", subcore_axis_name="s", num_cores=NC, num_subcores=NS
)


@functools.partial(
    pl.kernel,
    mesh=_mesh,
    out_type=jax.ShapeDtypeStruct((B * S * D,), jnp.float32),
    scratch_types=[
        pltpu.VMEM((CW,), jnp.float32),
        pltpu.VMEM((CW,), jnp.float32),
        pltpu.SemaphoreType.DMA,
        pltpu.SemaphoreType.DMA,
        pltpu.SemaphoreType.DMA,
    ],
)
def _sc_add(x_hbm, pos_hbm, out_hbm, bufx, bufp, sem_in, sem_p, sem_out):
    wid = lax.axis_index("s") * NC + lax.axis_index("c")
    s0 = wid * SPW

    def chunk(k, _):
        p_off = (s0 + k * C) * D
        pltpu.async_copy(pos_hbm.at[pl.ds(p_off, CW)], bufp, sem_p).wait()
        for b in range(B):
            x_off = b * S * D + p_off

            def addloop(e, _):
                base = e * (16 * U)
                for u in range(U):
                    i = base + u * 16
                    plsc.addupdate(bufx.at[pl.ds(i, 16)], bufp[pl.ds(i, 16)])
                return 0

            pltpu.async_copy(x_hbm.at[pl.ds(x_off, CW)], bufx, sem_in).wait()
            lax.fori_loop(0, CW // (16 * U), addloop, 0)
            pltpu.async_copy(bufx, out_hbm.at[pl.ds(x_off, CW)], sem_out).wait()
        return 0

    lax.fori_loop(0, NCHUNK, chunk, 0)


def kernel(x, pos_table):
    out = _sc_add(x.reshape(-1), pos_table.reshape(-1))
    return out.reshape(B, S, D)


# speedup vs baseline: 1.2307x; 1.2307x over previous
import functools

import jax
import jax.numpy as jnp
from jax import lax
from jax.experimental import pallas as pl
from jax.experimental.pallas import tpu as pltpu
from jax.experimental.pallas import tpu_sc as plsc

B, S, D = 4, 8192, 1024
NC, NS = 2, 16
NW = NC * NS
SPW = S // NW
C = 16
NCHUNK = SPW // C
CW = C * D
NB = 5
NT = NCHUNK * B
U = 8

_mesh = plsc.VectorSubcoreMesh(
    core_axis_name="c", subcore_axis_name="s", num_cores=NC, num_subcores=NS
)

_scratch = (
    [pltpu.VMEM((CW,), jnp.float32) for _ in range(NB)]
    + [pltpu.VMEM((CW,), jnp.float32) for _ in range(2)]
    + [pltpu.SemaphoreType.DMA for _ in range(NB + NB + 2)]
)


@functools.partial(
    pl.kernel,
    mesh=_mesh,
    out_type=jax.ShapeDtypeStruct((B * S * D,), jnp.float32),
    scratch_types=_scratch,
)
def _sc_add(x_hbm, pos_hbm, out_hbm, *refs):
    xb = refs[:NB]
    pb = refs[NB:NB + 2]
    semx = refs[NB + 2:2 * NB + 2]
    semo = refs[2 * NB + 2:3 * NB + 2]
    semp = refs[3 * NB + 2:]

    wid = lax.axis_index("s") * NC + lax.axis_index("c")
    s0 = wid * SPW

    def x_off(t):
        k, b = divmod(t, B)
        return b * S * D + (s0 + k * C) * D

    def load_x(t):
        return pltpu.async_copy(
            x_hbm.at[pl.ds(x_off(t), CW)], xb[t % NB], semx[t % NB]
        )

    def load_p(k):
        return pltpu.async_copy(
            pos_hbm.at[pl.ds((s0 + k * C) * D, CW)], pb[k % 2], semp[k % 2]
        )

    dx, do, dp = {}, {}, {}
    dp[0] = load_p(0)
    for t in range(3):
        dx[t] = load_x(t)

    for t in range(NT):
        k, b = divmod(t, B)
        if b == 0:
            dp[k].wait()
            if k + 1 < NCHUNK:
                dp[k + 1] = load_p(k + 1)
        tl = t + 3
        if tl < NT:
            if tl - NB >= 0:
                do[tl - NB].wait()
            dx[tl] = load_x(tl)
        dx[t].wait()

        xr, pr = xb[t % NB], pb[k % 2]

        def addloop(e, _):
            base = e * (16 * U)
            for u in range(U):
                i = base + u * 16
                plsc.addupdate(xr.at[pl.ds(i, 16)], pr[pl.ds(i, 16)])
            return 0

        lax.fori_loop(0, CW // (16 * U), addloop, 0)
        do[t] = pltpu.async_copy(
            xb[t % NB], out_hbm.at[pl.ds(x_off(t), CW)], semo[t % NB]
        )

    for t in range(NT - NB, NT):
        do[t].wait()


def kernel(x, pos_table):
    out = _sc_add(x.reshape(-1), pos_table.reshape(-1))
    return out.reshape(B, S, D)


# speedup vs baseline: 5.1493x; 4.1840x over previous
import jax
import jax.numpy as jnp
from jax.experimental import pallas as pl

B, S, D = 4, 8192, 1024
BS = 2048


def _body(x_ref, pos_ref, o_ref):
    o_ref[...] = x_ref[...] + pos_ref[...]


def kernel(x, pos_table):
    grid = (S // BS, B)
    return pl.pallas_call(
        _body,
        grid=grid,
        in_specs=[
            pl.BlockSpec((1, BS, D), lambda s, b: (b, s, 0)),
            pl.BlockSpec((BS, D), lambda s, b: (s, 0)),
        ],
        out_specs=pl.BlockSpec((1, BS, D), lambda s, b: (b, s, 0)),
        out_shape=jax.ShapeDtypeStruct((B, S, D), x.dtype),
    )(x, pos_table)
